# baseline (speedup 1.0000x reference)
import functools

import jax
import jax.numpy as jnp
from jax.experimental import pallas as pl
from jax.experimental.pallas import tpu as pltpu


def _round_up(x, m):
    return (x + m - 1) // m * m


def _mm_kernel(x_ref, w_ref, o_ref, *, nk, tk, k_tail):
    k = pl.program_id(0)

    def partial_dot(masked):
        x = x_ref[...]
        x = x.reshape(-1, x.shape[-1]) if x.ndim == 3 else x
        w = w_ref[...]
        if masked:
            xcol = jax.lax.broadcasted_iota(jnp.int32, x.shape, 1)
            wrow = jax.lax.broadcasted_iota(jnp.int32, w.shape, 0)
            x = jnp.where(xcol < k_tail, x, 0.0)
            w = jnp.where(wrow < k_tail, w, 0.0)
        out = jnp.dot(
            x.astype(jnp.bfloat16),
            w.astype(jnp.bfloat16),
            preferred_element_type=jnp.float32,
        )
        return out.reshape(o_ref.shape)

    @pl.when(k == 0)
    def _():
        o_ref[...] = partial_dot(masked=(nk == 1 and k_tail != tk))

    @pl.when(jnp.logical_and(k > 0, k < nk - 1))
    def _():
        o_ref[...] += partial_dot(masked=False)

    if nk > 1:
        @pl.when(k == nk - 1)
        def _():
            o_ref[...] += partial_dot(masked=(k_tail != tk))


def _k_range_matmul(x, w, B, S, Hp, tk, k_tail, x_3d):
    K = x.shape[-1]
    nk = -(-K // tk)
    if x_3d:
        x_spec = pl.BlockSpec((B, S, tk), lambda k: (0, 0, k))
    else:
        x_spec = pl.BlockSpec((B * S, tk), lambda k: (0, k))
    return pl.pallas_call(
        functools.partial(_mm_kernel, nk=nk, tk=tk, k_tail=k_tail),
        out_shape=jax.ShapeDtypeStruct((B, S, Hp), w.dtype),
        grid=(nk,),
        in_specs=[x_spec, pl.BlockSpec((tk, Hp), lambda k: (k, 0))],
        out_specs=pl.BlockSpec((B, S, Hp), lambda k: (0, 0, 0)),
        compiler_params=pltpu.CompilerParams(
            dimension_semantics=("arbitrary",),
            vmem_limit_bytes=100 * 1024 * 1024,
        ),
    )(x, w)


def kernel(weight, mask):
    B, S, V = mask.shape
    Vw, H = weight.shape
    M = B * S

    tk = 2048
    Hp = _round_up(H, 128)
    w = weight if Hp == H else jnp.pad(weight, ((0, 0), (0, Hp - H)))

    nk = -(-V // tk)
    nk1 = max(1, min(nk - 1, round(nk * 0.4)))
    V1 = nk1 * tk
    if V1 >= V:
        return _k_range_matmul(
            mask, w, B, S, Hp, tk, V - (nk - 1) * tk, x_3d=True
        )[..., :H] if Hp != H else _k_range_matmul(
            mask, w, B, S, Hp, tk, V - (nk - 1) * tk, x_3d=True)

    x2 = mask[:, :, V1:].reshape(M, V - V1)
    w2 = w[V1:, :]

    part1 = pl.pallas_call(
        functools.partial(_mm_kernel, nk=nk1, tk=tk, k_tail=tk),
        out_shape=jax.ShapeDtypeStruct((B, S, Hp), w.dtype),
        grid=(nk1,),
        in_specs=[
            pl.BlockSpec((B, S, tk), lambda k: (0, 0, k)),
            pl.BlockSpec((tk, Hp), lambda k: (k, 0)),
        ],
        out_specs=pl.BlockSpec((B, S, Hp), lambda k: (0, 0, 0)),
        compiler_params=pltpu.CompilerParams(
            dimension_semantics=("arbitrary",),
            vmem_limit_bytes=100 * 1024 * 1024,
        ),
    )(mask, w)

    K2 = V - V1
    nk2 = -(-K2 // tk)
    k_tail2 = K2 - (nk2 - 1) * tk
    part2 = _k_range_matmul(x2, w2, B, S, Hp, tk, k_tail2, x_3d=False)

    out = part1 + part2
    if Hp != H:
        out = out[..., :H]
    return out


# speedup vs baseline: 1.8373x; 1.8373x over previous
import functools

import jax
import jax.numpy as jnp
from jax.experimental import pallas as pl
from jax.experimental.pallas import tpu as pltpu


def _round_up(x, m):
    return (x + m - 1) // m * m


def _mm_kernel(x_ref, w_ref, o_ref, *, nk, tk, k_tail):
    k = pl.program_id(0)

    def partial_dot(masked):
        x = x_ref[...]
        w = w_ref[...]
        if masked:
            xcol = jax.lax.broadcasted_iota(jnp.int32, x.shape, 1)
            wrow = jax.lax.broadcasted_iota(jnp.int32, w.shape, 0)
            x = jnp.where(xcol < k_tail, x, 0.0)
            w = jnp.where(wrow < k_tail, w, 0.0)
        out = jnp.dot(
            x.astype(jnp.bfloat16),
            w.astype(jnp.bfloat16),
            preferred_element_type=jnp.float32,
        )
        return out.reshape(o_ref.shape)

    @pl.when(k == 0)
    def _():
        o_ref[...] = partial_dot(masked=(nk == 1 and k_tail != tk))

    @pl.when(jnp.logical_and(k > 0, k < nk - 1))
    def _():
        o_ref[...] += partial_dot(masked=False)

    if nk > 1:
        @pl.when(k == nk - 1)
        def _():
            o_ref[...] += partial_dot(masked=(k_tail != tk))


def kernel(weight, mask):
    B, S, V = mask.shape
    Vw, H = weight.shape
    M = B * S
    x = mask.reshape(M, V)

    Hp = _round_up(H, 128)
    w = weight if Hp == H else jnp.pad(weight, ((0, 0), (0, Hp - H)))

    tk = 2048
    nk = -(-V // tk)
    k_tail = V - (nk - 1) * tk

    out = pl.pallas_call(
        functools.partial(_mm_kernel, nk=nk, tk=tk, k_tail=k_tail),
        out_shape=jax.ShapeDtypeStruct((B, S, Hp), weight.dtype),
        grid=(nk,),
        in_specs=[
            pl.BlockSpec((M, tk), lambda k: (0, k)),
            pl.BlockSpec((tk, Hp), lambda k: (k, 0)),
        ],
        out_specs=pl.BlockSpec((B, S, Hp), lambda k: (0, 0, 0)),
        compiler_params=pltpu.CompilerParams(
            dimension_semantics=("arbitrary",),
            vmem_limit_bytes=100 * 1024 * 1024,
        ),
    )(x, w)
    return out[..., :H] if Hp != H else out
